# baseline (speedup 1.0000x reference)
import functools

import jax
import jax.numpy as jnp
from jax import lax
from jax.experimental import pallas as pl
from jax.experimental.pallas import tpu as pltpu
from jax.experimental.pallas import tpu_sc as plsc

N = 10000
NP = 10240
E = 320000
D = 128
ALPHA = 0.2

NW = 32
C = E // NW
CH = 80
NCH = C // CH
ER = E // CH
SEG = NP // 16
F32 = jnp.float32
I32 = jnp.int32

_mesh_cache = []


def _MESH():
    if not _mesh_cache:
        _mesh_cache.append(plsc.VectorSubcoreMesh(
            core_axis_name="c", subcore_axis_name="s"))
    return _mesh_cache[0]


def _node_body(x_ref, w_ref, a1_ref, a2_ref, h_ref, u1_ref, w2_ref):
    h = jnp.dot(x_ref[...], w_ref[...], preferred_element_type=F32)
    h_ref[...] = h
    u1_ref[...] = jnp.sum(h * a1_ref[0, :D][None, :], axis=1)
    w2_ref[...] = jnp.sum(h * a2_ref[0, D:][None, :], axis=1)


def _node_pre(x_pad, W, a1, a2):
    BM = 2048
    return pl.pallas_call(
        _node_body,
        grid=(NP // BM,),
        in_specs=[pl.BlockSpec((BM, D), lambda i: (i, 0)),
                  pl.BlockSpec((D, D), lambda i: (0, 0)),
                  pl.BlockSpec((1, 2 * D), lambda i: (0, 0)),
                  pl.BlockSpec((1, 2 * D), lambda i: (0, 0))],
        out_specs=[pl.BlockSpec((BM, D), lambda i: (i, 0)),
                   pl.BlockSpec((BM,), lambda i: (i,)),
                   pl.BlockSpec((BM,), lambda i: (i,))],
        out_shape=[jax.ShapeDtypeStruct((NP, D), F32),
                   jax.ShapeDtypeStruct((NP,), F32),
                   jax.ShapeDtypeStruct((NP,), F32)],
    )(x_pad, W, a1, a2)


EBM = 16000


def _edge_body(p_ref, nh_ref, w_ref, a1_ref, a2_ref, v1_ref, u2_ref):
    i = pl.program_id(0)
    W = w_ref[...]
    c1 = jnp.sum(W * a1_ref[0, D:][None, :], axis=1)
    c2 = jnp.sum(W * a2_ref[0, :D][None, :], axis=1)
    v1_ref[pl.ds(i * EBM, EBM)] = jnp.dot(p_ref[...], c1,
                                          preferred_element_type=F32)
    u2_ref[pl.ds(i * EBM, EBM)] = jnp.dot(nh_ref[...], c2,
                                          preferred_element_type=F32)


def _edge_pre(p_h, new_h, W, a1, a2):
    BM = EBM
    return pl.pallas_call(
        _edge_body,
        grid=(E // BM,),
        in_specs=[pl.BlockSpec((BM, D), lambda i: (i, 0)),
                  pl.BlockSpec((BM, D), lambda i: (i, 0)),
                  pl.BlockSpec((D, D), lambda i: (0, 0)),
                  pl.BlockSpec((1, 2 * D), lambda i: (0, 0)),
                  pl.BlockSpec((1, 2 * D), lambda i: (0, 0))],
        out_specs=[pl.BlockSpec((E,), lambda i: (0,)),
                   pl.BlockSpec((E,), lambda i: (0,))],
        out_shape=[jax.ShapeDtypeStruct((E,), F32),
                   jax.ShapeDtypeStruct((E,), F32)],
    )(p_h, new_h, W, a1, a2)


def _combine_body(p_ref, o_ref):
    hp = p_ref[...]
    o_ref[...] = jnp.where(hp > 0, hp, jnp.exp(jnp.minimum(hp, 0.0)) - 1.0)


def _combine(p):
    BM = 2000
    return pl.pallas_call(
        _combine_body,
        grid=(N // BM,),
        in_specs=[pl.BlockSpec((BM, D), lambda i: (i, 0))],
        out_specs=pl.BlockSpec((BM, D), lambda i: (i, 0)),
        out_shape=jax.ShapeDtypeStruct((N, D), F32),
    )(p)


def _worker():
    cid = lax.axis_index("c")
    sid = lax.axis_index("s")
    return cid, sid, cid * 16 + sid


def _leaky_exp(s):
    return jnp.exp(-jnp.where(s >= 0, s, ALPHA * s))


_GDN = lax.GatherDimensionNumbers(
    offset_dims=(), collapsed_slice_dims=(0,), start_index_map=(0,))


def _splat(vec, j):
    idx = jnp.full((16, 1), j, I32)
    return lax.gather(vec, idx, _GDN, slice_sizes=(1,),
                      mode=lax.GatherScatterMode.PROMISE_IN_BOUNDS)


def _sc_attn_col(u1, v1_2d, ec0_2d, row_2d):
    @functools.partial(
        pl.kernel,
        out_type=[jax.ShapeDtypeStruct((NW, NCH, CH), F32),
                  jax.ShapeDtypeStruct((2 * NP,), F32)],
        mesh=_MESH(),
        compiler_params=pltpu.CompilerParams(needs_layout_passes=False),
        scratch_types=[
            pltpu.VMEM((NP,), F32),
            pltpu.VMEM((NCH, CH), F32),
            pltpu.VMEM((NCH, CH), I32),
            pltpu.VMEM((NCH, CH), I32),
            pltpu.VMEM((NCH, CH), F32),
            pltpu.VMEM((SEG,), F32),
            pltpu.VMEM_SHARED((NP,), F32),
            pltpu.SemaphoreType.DMA,
        ],
    )
    def kern(u1_hbm, v1_hbm, ec0_hbm, row_hbm, ex1_hbm, part_hbm,
             u1_v, v1_v, ec0_v, row_v, ex1_v, zb_v, seg_sh, sem):
        cid, sid, wid = _worker()
        pltpu.sync_copy(u1_hbm, u1_v)
        pltpu.sync_copy(v1_hbm.at[wid], v1_v)
        pltpu.sync_copy(ec0_hbm.at[wid], ec0_v)
        pltpu.sync_copy(row_hbm.at[wid], row_v)

        def zfill(i, _):
            zb_v[pl.ds(i * 16, 16)] = jnp.zeros((16,), F32)
            return 0
        lax.fori_loop(0, SEG // 16, zfill, 0)
        pltpu.sync_copy(zb_v, seg_sh.at[pl.ds(sid * SEG, SEG)])
        plsc.subcore_barrier()

        def row_fn(r, _):
            for k in range(CH // 16):
                sl = pl.ds(k * 16, 16)
                g = plsc.load_gather(u1_v, [ec0_v[r, sl]])
                ec = _leaky_exp(g + v1_v[r, sl])
                ex1_v[r, sl] = jnp.exp(ec)
            return 0
        lax.fori_loop(0, NCH, row_fn, 0)
        pltpu.sync_copy(ex1_v, ex1_hbm.at[wid])

        def scat(w, _):
            cps = []
            for j in range(5):
                ch = w * 5 + j
                cps.append(pltpu.async_copy(
                    ex1_v.at[ch], seg_sh.at[row_v.at[ch]], sem, add=True))
            for cp in cps:
                cp.wait()
            return 0
        lax.fori_loop(0, NCH // 5, scat, 0)
        plsc.subcore_barrier()
        pltpu.sync_copy(seg_sh.at[pl.ds(sid * SEG, SEG)],
                        part_hbm.at[pl.ds(cid * NP + sid * SEG, SEG)])

    return kern(u1, v1_2d, ec0_2d, row_2d)


def _sc_softmax_div(seg_part, ex1_2d, row_2d):
    @functools.partial(
        pl.kernel,
        out_type=jax.ShapeDtypeStruct((NW, NCH, CH), F32),
        mesh=_MESH(),
        compiler_params=pltpu.CompilerParams(needs_layout_passes=False),
        scratch_types=[
            pltpu.VMEM((NP,), F32),
            pltpu.VMEM((NP,), F32),
            pltpu.VMEM((NCH, CH), F32),
            pltpu.VMEM((NCH, CH), I32),
            pltpu.VMEM((NCH, CH), F32),
        ],
    )
    def kern(part_hbm, ex1_hbm, row_hbm, z_hbm,
             pa_v, pb_v, ex1_v, row_v, z_v):
        cid, sid, wid = _worker()
        pltpu.sync_copy(part_hbm.at[pl.ds(0, NP)], pa_v)
        pltpu.sync_copy(part_hbm.at[pl.ds(NP, NP)], pb_v)
        pltpu.sync_copy(ex1_hbm.at[wid], ex1_v)
        pltpu.sync_copy(row_hbm.at[wid], row_v)

        def red(i, _):
            sl = pl.ds(i * 16, 16)
            pa_v[sl] = pa_v[sl] + pb_v[sl] + 1e-16
            return 0
        lax.fori_loop(0, NP // 16, red, 0)

        def row_fn(r, _):
            for k in range(CH // 16):
                sl = pl.ds(k * 16, 16)
                ss = plsc.load_gather(pa_v, [row_v[r, sl]])
                z_v[r, sl] = ex1_v[r, sl] / ss
            return 0
        lax.fori_loop(0, NCH, row_fn, 0)
        pltpu.sync_copy(z_v, z_hbm.at[wid])

    return kern(seg_part, ex1_2d, row_2d)


def _sc_edge_row(w2n, u2_flat, z_flat, rr_2d, e1_2d, e0_2d):
    @functools.partial(
        pl.kernel,
        out_type=[jax.ShapeDtypeStruct((NW, NCH, CH), F32),
                  jax.ShapeDtypeStruct((2 * NP,), F32)],
        mesh=_MESH(),
        compiler_params=pltpu.CompilerParams(needs_layout_passes=False),
        scratch_types=[
            pltpu.VMEM((NP,), F32),
            pltpu.VMEM((NCH, CH), I32),
            pltpu.VMEM((NCH, CH), I32),
            pltpu.VMEM((NCH, CH), I32),
            pltpu.VMEM((NCH, CH), F32),
            pltpu.VMEM((NCH, CH), F32),
            pltpu.VMEM((NCH, CH), F32),
            pltpu.VMEM((SEG,), F32),
            pltpu.VMEM_SHARED((NP,), F32),
            pltpu.SemaphoreType.DMA,
        ],
    )
    def kern(w2_hbm, u2_hbm, z_hbm, rr_hbm, e1_hbm, e0_hbm, ee_hbm, part_hbm,
             w2_v, rr_v, e1_v, e0_v, u2r_v, zr_v, ee_v, zb_v, rs_sh, sem):
        cid, sid, wid = _worker()
        pltpu.sync_copy(w2_hbm, w2_v)
        pltpu.sync_copy(rr_hbm.at[wid], rr_v)
        pltpu.sync_copy(e1_hbm.at[wid], e1_v)
        pltpu.sync_copy(e0_hbm.at[wid], e0_v)

        def zfill(i, _):
            zb_v[pl.ds(i * 16, 16)] = jnp.zeros((16,), F32)
            return 0
        lax.fori_loop(0, SEG // 16, zfill, 0)
        pltpu.sync_copy(zb_v, rs_sh.at[pl.ds(sid * SEG, SEG)])
        plsc.subcore_barrier()

        def gat(w, _):
            cps = []
            for j in range(5):
                ch = w * 5 + j
                cps.append(pltpu.async_copy(
                    u2_hbm.at[rr_v.at[ch]], u2r_v.at[ch], sem))
                cps.append(pltpu.async_copy(
                    z_hbm.at[rr_v.at[ch]], zr_v.at[ch], sem))
            for cp in cps:
                cp.wait()
            return 0
        lax.fori_loop(0, NCH // 5, gat, 0)

        def row_fn(r, _):
            for k in range(CH // 16):
                sl = pl.ds(k * 16, 16)
                g = plsc.load_gather(w2_v, [e1_v[r, sl]])
                er = _leaky_exp(u2r_v[r, sl] + g)
                ee_v[r, sl] = er * zr_v[r, sl]
            return 0
        lax.fori_loop(0, NCH, row_fn, 0)
        pltpu.sync_copy(ee_v, ee_hbm.at[wid])

        def scat(w, _):
            cps = []
            for j in range(5):
                ch = w * 5 + j
                cps.append(pltpu.async_copy(
                    ee_v.at[ch], rs_sh.at[e0_v.at[ch]], sem, add=True))
            for cp in cps:
                cp.wait()
            return 0
        lax.fori_loop(0, NCH // 5, scat, 0)
        plsc.subcore_barrier()
        pltpu.sync_copy(rs_sh.at[pl.ds(sid * SEG, SEG)],
                        part_hbm.at[pl.ds(cid * NP + sid * SEG, SEG)])

    return kern(w2n, u2_flat, z_flat, rr_2d, e1_2d, e0_2d)


HALF = NP // 2
GS2 = 24
GRP = GS2 * CH
CAPG = 6
CAP = CAPG * GRP


def _sc_partition(ee_2d, e0_2d, e1_2d, rs_part):
    @functools.partial(
        pl.kernel,
        out_type=[jax.ShapeDtypeStruct((NW, NCH, CH), F32),
                  jax.ShapeDtypeStruct((NW * 2 * CAPG, GS2, CH), I32),
                  jax.ShapeDtypeStruct((NW * 2 * CAPG, GS2, CH), I32),
                  jax.ShapeDtypeStruct((NW * 2 * CAPG, GS2, CH), F32),
                  jax.ShapeDtypeStruct((NW, 1, 16), I32)],
        mesh=_MESH(),
        compiler_params=pltpu.CompilerParams(needs_layout_passes=False),
        scratch_types=[
            pltpu.VMEM((NP,), F32),
            pltpu.VMEM((SEG,), F32),
            pltpu.VMEM((NCH, CH), F32),
            pltpu.VMEM((NCH, CH), I32),
            pltpu.VMEM((NCH, CH), I32),
            pltpu.VMEM((CAP,), I32),
            pltpu.VMEM((CAP,), I32),
            pltpu.VMEM((CAP,), F32),
            pltpu.VMEM((640,), I32),
            pltpu.VMEM((GS2, CH), I32),
            pltpu.VMEM((GS2, CH), I32),
            pltpu.VMEM((GS2, CH), F32),
            pltpu.VMEM((1, 16), I32),
        ],
    )
    def kern(ee_hbm, e0_hbm, e1_hbm, part_hbm, att_hbm, bk0_hbm, bk1_hbm,
             bka_hbm, cnt_hbm,
             rs_v, tmp_v, ee_v, e0_v, e1_v, f0_v, f1_v, fa_v, pc_v,
             o0_v, o1_v, oa_v, counts_v):
        cid, sid, wid = _worker()
        pltpu.sync_copy(part_hbm.at[pl.ds(0, NP)], rs_v)

        def red(k, _):
            pltpu.sync_copy(part_hbm.at[pl.ds(NP + k * SEG, SEG)], tmp_v)

            def red2(j, _):
                sl = pl.ds(j * 16, 16)
                gl = pl.ds(k * SEG + j * 16, 16)
                t = rs_v[gl] + tmp_v[sl]
                rs_v[gl] = jnp.where(t == 0, 1.0, t)
                return 0
            lax.fori_loop(0, SEG // 16, red2, 0)
            return 0
        lax.fori_loop(0, 16, red, 0)

        pltpu.sync_copy(ee_hbm.at[wid], ee_v)
        pltpu.sync_copy(e0_hbm.at[wid], e0_v)
        pltpu.sync_copy(e1_hbm.at[wid], e1_v)

        def att_fn(r, _):
            for k in range(CH // 16):
                sl = pl.ds(k * 16, 16)
                rs = plsc.load_gather(rs_v, [e0_v[r, sl]])
                ee_v[r, sl] = ee_v[r, sl] / rs
            return 0
        lax.fori_loop(0, NCH, att_fn, 0)
        pltpu.sync_copy(ee_v, att_hbm.at[wid])

        ngrp_vecs = []
        for hf in range(2):
            base = hf * HALF

            def zf(i, _):
                sl = pl.ds(i * 16, 16)
                zi = jnp.zeros((16,), I32)
                f0_v[sl] = zi
                f1_v[sl] = zi
                fa_v[sl] = jnp.zeros((16,), F32)
                return 0
            lax.fori_loop(0, CAP // 16, zf, 0)

            def zp(i, _):
                pc_v[pl.ds(i * 16, 16)] = jnp.zeros((16,), I32)
                return 0
            lax.fori_loop(0, 640 // 16, zp, 0)

            def mkmask(r, k):
                sl = pl.ds(k * 16, 16)
                loc = e0_v[r, sl] - base
                return loc, (loc >= 0) & (loc < HALF)

            def cpass(r, _):
                for k in range(CH // 16):
                    _, m = mkmask(r, k)
                    c = plsc.all_reduce_population_count(m)
                    i = r * (CH // 16) + k
                    al = pl.ds((i // 16) * 16, 16)
                    old = pc_v[al]
                    pc_v[al] = jnp.where(lax.iota(I32, 16) == i % 16, c, old)
                return 0
            lax.fori_loop(0, NCH, cpass, 0)

            def ppass(v, carry):
                cnt16 = pc_v[pl.ds(v * 16, 16)]
                cum = plsc.cumsum(cnt16)
                pc_v[pl.ds(v * 16, 16)] = carry + cum - cnt16
                return carry + _splat(cum, 15)
            NV = NCH * (CH // 16)
            total = lax.fori_loop(0, (NV + 15) // 16, ppass,
                                  jnp.zeros((16,), I32))

            def spass(r, _):
                for k in range(CH // 16):
                    loc, m = mkmask(r, k)
                    i = r * (CH // 16) + k
                    off = _splat(pc_v[pl.ds((i // 16) * 16, 16)], i % 16)
                    sl = pl.ds(k * 16, 16)
                    pos = off + plsc.cumsum(m.astype(I32)) - 1
                    pos = jnp.where(m, pos, 0)
                    plsc.store_scatter(f0_v, [pos], loc, mask=m)
                    plsc.store_scatter(f1_v, [pos], e1_v[r, sl], mask=m)
                    plsc.store_scatter(fa_v, [pos], ee_v[r, sl], mask=m)
                return 0
            lax.fori_loop(0, NCH, spass, 0)
            ngrp_vecs.append((total + (GRP - 1)) // GRP)

            def wgrp(g, _):
                def wrow(r, _):
                    for k in range(CH // 16):
                        sl = pl.ds(k * 16, 16)
                        off = pl.ds(g * GRP + r * CH + k * 16, 16)
                        o0_v[r, sl] = f0_v[off]
                        o1_v[r, sl] = f1_v[off]
                        oa_v[r, sl] = fa_v[off]
                    return 0
                lax.fori_loop(0, GS2, wrow, 0)
                j = (wid * 2 + hf) * CAPG + g
                pltpu.sync_copy(o0_v, bk0_hbm.at[j])
                pltpu.sync_copy(o1_v, bk1_hbm.at[j])
                pltpu.sync_copy(oa_v, bka_hbm.at[j])
                return 0
            lax.fori_loop(0, CAPG, wgrp, 0)
        lanes = lax.iota(I32, 16)
        counts_v[0, pl.ds(0, 16)] = jnp.where(
            lanes == 0, _splat(ngrp_vecs[0], 0),
            jnp.where(lanes == 1, _splat(ngrp_vecs[1], 0), 0))
        pltpu.sync_copy(counts_v, cnt_hbm.at[wid])

    return kern(ee_2d, e0_2d, e1_2d, rs_part)


def _sc_spmm(h_pad, bk0, bk1, bka, counts):
    SEGH = HALF // 16

    @functools.partial(
        pl.kernel,
        out_type=jax.ShapeDtypeStruct((2, HALF, D), F32),
        mesh=_MESH(),
        compiler_params=pltpu.CompilerParams(needs_layout_passes=False),
        scratch_types=[
            pltpu.VMEM((1, 16), I32),
            pltpu.VMEM((GS2, CH), I32),
            pltpu.VMEM((GS2, CH), I32),
            pltpu.VMEM((GS2, CH), F32),
            pltpu.VMEM((CH, D), F32),
            pltpu.VMEM((CH, D), F32),
            pltpu.VMEM((CH, D), F32),
            pltpu.VMEM_SHARED((HALF, D), F32),
            pltpu.SemaphoreType.DMA,
            pltpu.SemaphoreType.DMA,
        ],
    )
    def kern(h_hbm, bk0_hbm, bk1_hbm, bka_hbm, cnt_hbm, hp_hbm,
             cv, ge0, ge1, gatt, ga_v, gb_v, sb_v, hp_sh, sg, ss):
        cid, sid, wid = _worker()

        for i in range(16):
            for k in range(D // 16):
                ga_v[i, pl.ds(k * 16, 16)] = jnp.zeros((16,), F32)

        def zrow(j, _):
            pltpu.sync_copy(ga_v.at[pl.ds(0, 16)],
                            hp_sh.at[pl.ds(sid * SEGH + j * 16, 16)])
            return 0
        lax.fori_loop(0, SEGH // 16, zrow, 0)
        plsc.subcore_barrier()

        def scale(ch, src, dst):
            def grp(g, _):
                av = gatt[ch, pl.ds(g * 16, 16)]
                for j in range(16):
                    spl = _splat(av, j)
                    r = g * 16 + j
                    for k in range(D // 16):
                        sl = pl.ds(k * 16, 16)
                        dst[r, sl] = src[r, sl] * spl
                return 0
            lax.fori_loop(0, CH // 16, grp, 0)

        def drain_gather(dst):
            pltpu.make_async_copy(h_hbm.at[pl.ds(0, CH)], dst, sg).wait()

        def drain_scatter():
            pltpu.make_async_copy(sb_v, hp_sh.at[pl.ds(0, CH)], ss).wait()

        for t_off in range(2):
            t = sid * 2 + t_off
            pltpu.sync_copy(cnt_hbm.at[t], cv)
            ngrp = _splat(cv[0, pl.ds(0, 16)], cid)[0]

            def bgrp(g, _):
                j = (t * 2 + cid) * CAPG + g
                pltpu.sync_copy(bk0_hbm.at[j], ge0)
                pltpu.sync_copy(bk1_hbm.at[j], ge1)
                pltpu.sync_copy(bka_hbm.at[j], gatt)

                pltpu.async_copy(h_hbm.at[ge1.at[0]], ga_v, sg)

                def chunk_pair(p2, _):
                    for par in range(2):
                        ch = p2 * 2 + par
                        gbuf = ga_v if par == 0 else gb_v
                        nxt = gb_v if par == 0 else ga_v
                        nch = jnp.where(ch + 1 < GS2, ch + 1, 0)
                        pltpu.async_copy(h_hbm.at[ge1.at[nch]], nxt, sg)
                        drain_gather(gbuf)

                        @pl.when(ch >= 1)
                        def _():
                            drain_scatter()
                        scale(ch, gbuf, sb_v)
                        pltpu.async_copy(sb_v, hp_sh.at[ge0.at[ch]], ss,
                                         add=True)
                    return 0
                lax.fori_loop(0, GS2 // 2, chunk_pair, 0)
                drain_gather(ga_v)
                drain_scatter()
                return 0
            lax.fori_loop(0, ngrp, bgrp, 0)
        plsc.subcore_barrier()
        pltpu.sync_copy(hp_sh.at[pl.ds(sid * SEGH, SEGH)],
                        hp_hbm.at[cid, pl.ds(sid * SEGH, SEGH)])

    return kern(h_pad, bk0, bk1, bka, counts)


def kernel(input, adj, edge, p_h, edge_col, row_i, row_resort, new_h, W, a1, a2):
    x_pad = jnp.pad(input, ((0, NP - N), (0, 0)))
    h_pad, u1, w2n = _node_pre(x_pad, W, a1, a2)
    v1, u2 = _edge_pre(p_h, new_h, W, a1, a2)

    ec0_2d = edge_col[0].reshape(NW, NCH, CH)
    row_2d = row_i.reshape(NW, NCH, CH)
    rr_2d = row_resort.reshape(NW, NCH, CH)
    e0_2d = edge[0].reshape(NW, NCH, CH)
    e1_2d = edge[1].reshape(NW, NCH, CH)

    ex1_2d, seg_part = _sc_attn_col(u1, v1.reshape(NW, NCH, CH), ec0_2d, row_2d)
    z_2d = _sc_softmax_div(seg_part, ex1_2d, row_2d)
    ee_2d, rs_part = _sc_edge_row(w2n, u2, z_2d.reshape(E), rr_2d, e1_2d, e0_2d)
    att_2d, bk0, bk1, bka, counts = _sc_partition(ee_2d, e0_2d, e1_2d,
                                                  rs_part)
    hp_halves = _sc_spmm(h_pad, bk0, bk1, bka, counts)

    h_prime = _combine(hp_halves.reshape(NP, D)[:N])
    return h_prime, edge, att_2d.reshape(E, 1)


# speedup vs baseline: 3.1656x; 3.1656x over previous
import functools

import jax
import jax.numpy as jnp
from jax import lax
from jax.experimental import pallas as pl
from jax.experimental.pallas import tpu as pltpu
from jax.experimental.pallas import tpu_sc as plsc

N = 10000
NP = 10240
E = 320000
D = 128
ALPHA = 0.2

NW = 32
C = E // NW
CH = 80
NCH = C // CH
ER = E // CH
SEG = NP // 16
F32 = jnp.float32
I32 = jnp.int32

_mesh_cache = []


def _MESH():
    if not _mesh_cache:
        _mesh_cache.append(plsc.VectorSubcoreMesh(
            core_axis_name="c", subcore_axis_name="s"))
    return _mesh_cache[0]


def _node_body(x_ref, w_ref, a1_ref, a2_ref, h_ref, u1_ref, w2_ref):
    h = jnp.dot(x_ref[...], w_ref[...], preferred_element_type=F32)
    h_ref[...] = h
    u1_ref[...] = jnp.sum(h * a1_ref[0, :D][None, :], axis=1)
    w2_ref[...] = jnp.sum(h * a2_ref[0, D:][None, :], axis=1)


def _node_pre(x_pad, W, a1, a2):
    BM = 2048
    return pl.pallas_call(
        _node_body,
        grid=(NP // BM,),
        in_specs=[pl.BlockSpec((BM, D), lambda i: (i, 0)),
                  pl.BlockSpec((D, D), lambda i: (0, 0)),
                  pl.BlockSpec((1, 2 * D), lambda i: (0, 0)),
                  pl.BlockSpec((1, 2 * D), lambda i: (0, 0))],
        out_specs=[pl.BlockSpec((BM, D), lambda i: (i, 0)),
                   pl.BlockSpec((BM,), lambda i: (i,)),
                   pl.BlockSpec((BM,), lambda i: (i,))],
        out_shape=[jax.ShapeDtypeStruct((NP, D), F32),
                   jax.ShapeDtypeStruct((NP,), F32),
                   jax.ShapeDtypeStruct((NP,), F32)],
    )(x_pad, W, a1, a2)


EBM = 16000


def _edge_body(p_ref, nh_ref, w_ref, a1_ref, a2_ref, v1_ref, u2_ref):
    i = pl.program_id(0)
    W = w_ref[...]
    c1 = jnp.sum(W * a1_ref[0, D:][None, :], axis=1)
    c2 = jnp.sum(W * a2_ref[0, :D][None, :], axis=1)
    v1_ref[pl.ds(i * EBM, EBM)] = jnp.dot(p_ref[...], c1,
                                          preferred_element_type=F32)
    u2_ref[pl.ds(i * EBM, EBM)] = jnp.dot(nh_ref[...], c2,
                                          preferred_element_type=F32)


def _edge_pre(p_h, new_h, W, a1, a2):
    BM = EBM
    return pl.pallas_call(
        _edge_body,
        grid=(E // BM,),
        in_specs=[pl.BlockSpec((BM, D), lambda i: (i, 0)),
                  pl.BlockSpec((BM, D), lambda i: (i, 0)),
                  pl.BlockSpec((D, D), lambda i: (0, 0)),
                  pl.BlockSpec((1, 2 * D), lambda i: (0, 0)),
                  pl.BlockSpec((1, 2 * D), lambda i: (0, 0))],
        out_specs=[pl.BlockSpec((E,), lambda i: (0,)),
                   pl.BlockSpec((E,), lambda i: (0,))],
        out_shape=[jax.ShapeDtypeStruct((E,), F32),
                   jax.ShapeDtypeStruct((E,), F32)],
    )(p_h, new_h, W, a1, a2)


def _combine_body(p_ref, o_ref):
    hp = p_ref[...]
    o_ref[...] = jnp.where(hp > 0, hp, jnp.exp(jnp.minimum(hp, 0.0)) - 1.0)


def _combine(p):
    BM = 2000
    return pl.pallas_call(
        _combine_body,
        grid=(N // BM,),
        in_specs=[pl.BlockSpec((BM, D), lambda i: (i, 0))],
        out_specs=pl.BlockSpec((BM, D), lambda i: (i, 0)),
        out_shape=jax.ShapeDtypeStruct((N, D), F32),
    )(p)


def _worker():
    cid = lax.axis_index("c")
    sid = lax.axis_index("s")
    return cid, sid, cid * 16 + sid


def _leaky_exp(s):
    return jnp.exp(-jnp.where(s >= 0, s, ALPHA * s))


_GDN = lax.GatherDimensionNumbers(
    offset_dims=(), collapsed_slice_dims=(0,), start_index_map=(0,))


def _splat(vec, j):
    idx = jnp.full((16, 1), j, I32)
    return lax.gather(vec, idx, _GDN, slice_sizes=(1,),
                      mode=lax.GatherScatterMode.PROMISE_IN_BOUNDS)


def _sc_attn_col(u1, v1_2d, ec0_2d, row_2d):
    @functools.partial(
        pl.kernel,
        out_type=[jax.ShapeDtypeStruct((NW, NCH, CH), F32),
                  jax.ShapeDtypeStruct((2 * NP,), F32)],
        mesh=_MESH(),
        compiler_params=pltpu.CompilerParams(needs_layout_passes=False),
        scratch_types=[
            pltpu.VMEM((NP,), F32),
            pltpu.VMEM((NCH, CH), F32),
            pltpu.VMEM((NCH, CH), I32),
            pltpu.VMEM((NCH, CH), I32),
            pltpu.VMEM((NCH, CH), F32),
            pltpu.VMEM((SEG,), F32),
            pltpu.VMEM_SHARED((NP,), F32),
            pltpu.SemaphoreType.DMA,
        ],
    )
    def kern(u1_hbm, v1_hbm, ec0_hbm, row_hbm, ex1_hbm, part_hbm,
             u1_v, v1_v, ec0_v, row_v, ex1_v, zb_v, seg_sh, sem):
        cid, sid, wid = _worker()
        pltpu.sync_copy(u1_hbm, u1_v)
        pltpu.sync_copy(v1_hbm.at[wid], v1_v)
        pltpu.sync_copy(ec0_hbm.at[wid], ec0_v)
        pltpu.sync_copy(row_hbm.at[wid], row_v)

        def zfill(i, _):
            zb_v[pl.ds(i * 16, 16)] = jnp.zeros((16,), F32)
            return 0
        lax.fori_loop(0, SEG // 16, zfill, 0)
        pltpu.sync_copy(zb_v, seg_sh.at[pl.ds(sid * SEG, SEG)])
        plsc.subcore_barrier()

        def row_fn(r, _):
            for k in range(CH // 16):
                sl = pl.ds(k * 16, 16)
                g = plsc.load_gather(u1_v, [ec0_v[r, sl]])
                ec = _leaky_exp(g + v1_v[r, sl])
                ex1_v[r, sl] = jnp.exp(ec)
            return 0
        lax.fori_loop(0, NCH, row_fn, 0)
        pltpu.sync_copy(ex1_v, ex1_hbm.at[wid])

        def scat(w, _):
            cps = []
            for j in range(5):
                ch = w * 5 + j
                cps.append(pltpu.async_copy(
                    ex1_v.at[ch], seg_sh.at[row_v.at[ch]], sem, add=True))
            for cp in cps:
                cp.wait()
            return 0
        lax.fori_loop(0, NCH // 5, scat, 0)
        plsc.subcore_barrier()
        pltpu.sync_copy(seg_sh.at[pl.ds(sid * SEG, SEG)],
                        part_hbm.at[pl.ds(cid * NP + sid * SEG, SEG)])

    return kern(u1, v1_2d, ec0_2d, row_2d)


def _sc_softmax_div(seg_part, ex1_2d, row_2d):
    @functools.partial(
        pl.kernel,
        out_type=jax.ShapeDtypeStruct((NW, NCH, CH), F32),
        mesh=_MESH(),
        compiler_params=pltpu.CompilerParams(needs_layout_passes=False),
        scratch_types=[
            pltpu.VMEM((NP,), F32),
            pltpu.VMEM((NP,), F32),
            pltpu.VMEM((NCH, CH), F32),
            pltpu.VMEM((NCH, CH), I32),
            pltpu.VMEM((NCH, CH), F32),
        ],
    )
    def kern(part_hbm, ex1_hbm, row_hbm, z_hbm,
             pa_v, pb_v, ex1_v, row_v, z_v):
        cid, sid, wid = _worker()
        pltpu.sync_copy(part_hbm.at[pl.ds(0, NP)], pa_v)
        pltpu.sync_copy(part_hbm.at[pl.ds(NP, NP)], pb_v)
        pltpu.sync_copy(ex1_hbm.at[wid], ex1_v)
        pltpu.sync_copy(row_hbm.at[wid], row_v)

        def red(i, _):
            sl = pl.ds(i * 16, 16)
            pa_v[sl] = pa_v[sl] + pb_v[sl] + 1e-16
            return 0
        lax.fori_loop(0, NP // 16, red, 0)

        def row_fn(r, _):
            for k in range(CH // 16):
                sl = pl.ds(k * 16, 16)
                ss = plsc.load_gather(pa_v, [row_v[r, sl]])
                z_v[r, sl] = ex1_v[r, sl] / ss
            return 0
        lax.fori_loop(0, NCH, row_fn, 0)
        pltpu.sync_copy(z_v, z_hbm.at[wid])

    return kern(seg_part, ex1_2d, row_2d)


def _sc_edge_row(w2n, u2_flat, z_flat, rr_2d, e1_2d, e0_2d):
    @functools.partial(
        pl.kernel,
        out_type=[jax.ShapeDtypeStruct((NW, NCH, CH), F32),
                  jax.ShapeDtypeStruct((2 * NP,), F32)],
        mesh=_MESH(),
        compiler_params=pltpu.CompilerParams(needs_layout_passes=False),
        scratch_types=[
            pltpu.VMEM((NP,), F32),
            pltpu.VMEM((NCH, CH), I32),
            pltpu.VMEM((NCH, CH), I32),
            pltpu.VMEM((NCH, CH), I32),
            pltpu.VMEM((NCH, CH), F32),
            pltpu.VMEM((NCH, CH), F32),
            pltpu.VMEM((NCH, CH), F32),
            pltpu.VMEM((SEG,), F32),
            pltpu.VMEM_SHARED((NP,), F32),
            pltpu.SemaphoreType.DMA,
        ],
    )
    def kern(w2_hbm, u2_hbm, z_hbm, rr_hbm, e1_hbm, e0_hbm, ee_hbm, part_hbm,
             w2_v, rr_v, e1_v, e0_v, u2r_v, zr_v, ee_v, zb_v, rs_sh, sem):
        cid, sid, wid = _worker()
        pltpu.sync_copy(w2_hbm, w2_v)
        pltpu.sync_copy(rr_hbm.at[wid], rr_v)
        pltpu.sync_copy(e1_hbm.at[wid], e1_v)
        pltpu.sync_copy(e0_hbm.at[wid], e0_v)

        def zfill(i, _):
            zb_v[pl.ds(i * 16, 16)] = jnp.zeros((16,), F32)
            return 0
        lax.fori_loop(0, SEG // 16, zfill, 0)
        pltpu.sync_copy(zb_v, rs_sh.at[pl.ds(sid * SEG, SEG)])
        plsc.subcore_barrier()

        def gat(w, _):
            cps = []
            for j in range(5):
                ch = w * 5 + j
                cps.append(pltpu.async_copy(
                    u2_hbm.at[rr_v.at[ch]], u2r_v.at[ch], sem))
                cps.append(pltpu.async_copy(
                    z_hbm.at[rr_v.at[ch]], zr_v.at[ch], sem))
            for cp in cps:
                cp.wait()
            return 0
        lax.fori_loop(0, NCH // 5, gat, 0)

        def row_fn(r, _):
            for k in range(CH // 16):
                sl = pl.ds(k * 16, 16)
                g = plsc.load_gather(w2_v, [e1_v[r, sl]])
                er = _leaky_exp(u2r_v[r, sl] + g)
                ee_v[r, sl] = er * zr_v[r, sl]
            return 0
        lax.fori_loop(0, NCH, row_fn, 0)
        pltpu.sync_copy(ee_v, ee_hbm.at[wid])

        def scat(w, _):
            cps = []
            for j in range(5):
                ch = w * 5 + j
                cps.append(pltpu.async_copy(
                    ee_v.at[ch], rs_sh.at[e0_v.at[ch]], sem, add=True))
            for cp in cps:
                cp.wait()
            return 0
        lax.fori_loop(0, NCH // 5, scat, 0)
        plsc.subcore_barrier()
        pltpu.sync_copy(rs_sh.at[pl.ds(sid * SEG, SEG)],
                        part_hbm.at[pl.ds(cid * NP + sid * SEG, SEG)])

    return kern(w2n, u2_flat, z_flat, rr_2d, e1_2d, e0_2d)


HALF = NP // 2
GS2 = 24
GRP = GS2 * CH
CAPG = 6
CAP = CAPG * GRP


def _sc_partition(ee_2d, e0_2d, e1_2d, rs_part):
    @functools.partial(
        pl.kernel,
        out_type=[jax.ShapeDtypeStruct((NW, NCH, CH), F32),
                  jax.ShapeDtypeStruct((NW * 2 * CAPG, GS2, CH), I32),
                  jax.ShapeDtypeStruct((NW * 2 * CAPG, GS2, CH), I32),
                  jax.ShapeDtypeStruct((NW * 2 * CAPG, GS2, CH), F32),
                  jax.ShapeDtypeStruct((NW, 1, 16), I32)],
        mesh=_MESH(),
        compiler_params=pltpu.CompilerParams(needs_layout_passes=False),
        scratch_types=[
            pltpu.VMEM((NP,), F32),
            pltpu.VMEM((SEG,), F32),
            pltpu.VMEM((NCH, CH), F32),
            pltpu.VMEM((NCH, CH), I32),
            pltpu.VMEM((NCH, CH), I32),
            pltpu.VMEM((CAP,), I32),
            pltpu.VMEM((CAP,), I32),
            pltpu.VMEM((CAP,), F32),
            pltpu.VMEM((640,), I32),
            pltpu.VMEM((GS2, CH), I32),
            pltpu.VMEM((GS2, CH), I32),
            pltpu.VMEM((GS2, CH), F32),
            pltpu.VMEM((1, 16), I32),
        ],
    )
    def kern(ee_hbm, e0_hbm, e1_hbm, part_hbm, att_hbm, bk0_hbm, bk1_hbm,
             bka_hbm, cnt_hbm,
             rs_v, tmp_v, ee_v, e0_v, e1_v, f0_v, f1_v, fa_v, pc_v,
             o0_v, o1_v, oa_v, counts_v):
        cid, sid, wid = _worker()
        pltpu.sync_copy(part_hbm.at[pl.ds(0, NP)], rs_v)

        def red(k, _):
            pltpu.sync_copy(part_hbm.at[pl.ds(NP + k * SEG, SEG)], tmp_v)

            def red2(j, _):
                sl = pl.ds(j * 16, 16)
                gl = pl.ds(k * SEG + j * 16, 16)
                t = rs_v[gl] + tmp_v[sl]
                rs_v[gl] = jnp.where(t == 0, 1.0, t)
                return 0
            lax.fori_loop(0, SEG // 16, red2, 0)
            return 0
        lax.fori_loop(0, 16, red, 0)

        pltpu.sync_copy(ee_hbm.at[wid], ee_v)
        pltpu.sync_copy(e0_hbm.at[wid], e0_v)
        pltpu.sync_copy(e1_hbm.at[wid], e1_v)

        def att_fn(r, _):
            for k in range(CH // 16):
                sl = pl.ds(k * 16, 16)
                rs = plsc.load_gather(rs_v, [e0_v[r, sl]])
                ee_v[r, sl] = ee_v[r, sl] / rs
            return 0
        lax.fori_loop(0, NCH, att_fn, 0)
        pltpu.sync_copy(ee_v, att_hbm.at[wid])

        ngrp_vecs = []
        for hf in range(2):
            base = hf * HALF

            def zf(i, _):
                sl = pl.ds(i * 16, 16)
                spread = i * 16 + lax.iota(I32, 16)
                f0_v[sl] = lax.rem(spread, jnp.int32(HALF))
                f1_v[sl] = lax.rem(spread, jnp.int32(N))
                fa_v[sl] = jnp.zeros((16,), F32)
                return 0
            lax.fori_loop(0, CAP // 16, zf, 0)

            def zp(i, _):
                pc_v[pl.ds(i * 16, 16)] = jnp.zeros((16,), I32)
                return 0
            lax.fori_loop(0, 640 // 16, zp, 0)

            def mkmask(r, k):
                sl = pl.ds(k * 16, 16)
                loc = e0_v[r, sl] - base
                return loc, (loc >= 0) & (loc < HALF)

            def cpass(r, _):
                for k in range(CH // 16):
                    _, m = mkmask(r, k)
                    c = plsc.all_reduce_population_count(m)
                    i = r * (CH // 16) + k
                    al = pl.ds((i // 16) * 16, 16)
                    old = pc_v[al]
                    pc_v[al] = jnp.where(lax.iota(I32, 16) == i % 16, c, old)
                return 0
            lax.fori_loop(0, NCH, cpass, 0)

            def ppass(v, carry):
                cnt16 = pc_v[pl.ds(v * 16, 16)]
                cum = plsc.cumsum(cnt16)
                pc_v[pl.ds(v * 16, 16)] = carry + cum - cnt16
                return carry + _splat(cum, 15)
            NV = NCH * (CH // 16)
            total = lax.fori_loop(0, (NV + 15) // 16, ppass,
                                  jnp.zeros((16,), I32))

            def spass(r, _):
                for k in range(CH // 16):
                    loc, m = mkmask(r, k)
                    i = r * (CH // 16) + k
                    off = _splat(pc_v[pl.ds((i // 16) * 16, 16)], i % 16)
                    sl = pl.ds(k * 16, 16)
                    pos = off + plsc.cumsum(m.astype(I32)) - 1
                    pos = jnp.where(m, pos, 0)
                    plsc.store_scatter(f0_v, [pos], loc, mask=m)
                    plsc.store_scatter(f1_v, [pos], e1_v[r, sl], mask=m)
                    plsc.store_scatter(fa_v, [pos], ee_v[r, sl], mask=m)
                return 0
            lax.fori_loop(0, NCH, spass, 0)
            ngrp_vecs.append((total + (GRP - 1)) // GRP)

            def wgrp(g, _):
                def wrow(r, _):
                    for k in range(CH // 16):
                        sl = pl.ds(k * 16, 16)
                        off = pl.ds(g * GRP + r * CH + k * 16, 16)
                        o0_v[r, sl] = f0_v[off]
                        o1_v[r, sl] = f1_v[off]
                        oa_v[r, sl] = fa_v[off]
                    return 0
                lax.fori_loop(0, GS2, wrow, 0)
                j = (wid * 2 + hf) * CAPG + g
                pltpu.sync_copy(o0_v, bk0_hbm.at[j])
                pltpu.sync_copy(o1_v, bk1_hbm.at[j])
                pltpu.sync_copy(oa_v, bka_hbm.at[j])
                return 0
            lax.fori_loop(0, CAPG, wgrp, 0)
        lanes = lax.iota(I32, 16)
        counts_v[0, pl.ds(0, 16)] = jnp.where(
            lanes == 0, _splat(ngrp_vecs[0], 0),
            jnp.where(lanes == 1, _splat(ngrp_vecs[1], 0), 0))
        pltpu.sync_copy(counts_v, cnt_hbm.at[wid])

    return kern(ee_2d, e0_2d, e1_2d, rs_part)


def _sc_spmm(h_pad, bk0, bk1, bka, counts):
    SEGH = HALF // 16

    @functools.partial(
        pl.kernel,
        out_type=jax.ShapeDtypeStruct((2, HALF, D), F32),
        mesh=_MESH(),
        compiler_params=pltpu.CompilerParams(needs_layout_passes=False),
        scratch_types=[
            pltpu.VMEM((1, 16), I32),
            pltpu.VMEM((GS2, CH), I32),
            pltpu.VMEM((GS2, CH), I32),
            pltpu.VMEM((GS2, CH), F32),
            pltpu.VMEM((CH, D), F32),
            pltpu.VMEM((CH, D), F32),
            pltpu.VMEM((CH, D), F32),
            pltpu.VMEM_SHARED((HALF, D), F32),
            pltpu.SemaphoreType.DMA,
            pltpu.SemaphoreType.DMA,
        ],
    )
    def kern(h_hbm, bk0_hbm, bk1_hbm, bka_hbm, cnt_hbm, hp_hbm,
             cv, ge0, ge1, gatt, ga_v, gb_v, sb_v, hp_sh, sg, ss):
        cid, sid, wid = _worker()

        for i in range(16):
            for k in range(D // 16):
                ga_v[i, pl.ds(k * 16, 16)] = jnp.zeros((16,), F32)

        def zrow(j, _):
            pltpu.sync_copy(ga_v.at[pl.ds(0, 16)],
                            hp_sh.at[pl.ds(sid * SEGH + j * 16, 16)])
            return 0
        lax.fori_loop(0, SEGH // 16, zrow, 0)
        plsc.subcore_barrier()

        def scale(ch, src, dst):
            def grp(g, _):
                av = gatt[ch, pl.ds(g * 16, 16)]
                for j in range(16):
                    spl = _splat(av, j)
                    r = g * 16 + j
                    for k in range(D // 16):
                        sl = pl.ds(k * 16, 16)
                        dst[r, sl] = src[r, sl] * spl
                return 0
            lax.fori_loop(0, CH // 16, grp, 0)

        def drain_gather(dst):
            pltpu.make_async_copy(h_hbm.at[pl.ds(0, CH)], dst, sg).wait()

        def drain_scatter():
            pltpu.make_async_copy(sb_v, hp_sh.at[pl.ds(0, CH)], ss).wait()

        for t_off in range(2):
            t = sid * 2 + t_off
            pltpu.sync_copy(cnt_hbm.at[t], cv)
            ngrp = _splat(cv[0, pl.ds(0, 16)], cid)[0]

            def bgrp(g, _):
                j = (t * 2 + cid) * CAPG + g
                pltpu.sync_copy(bk0_hbm.at[j], ge0)
                pltpu.sync_copy(bk1_hbm.at[j], ge1)
                pltpu.sync_copy(bka_hbm.at[j], gatt)

                pltpu.async_copy(h_hbm.at[ge1.at[0]], ga_v, sg)

                def chunk_pair(p2, _):
                    for par in range(2):
                        ch = p2 * 2 + par
                        gbuf = ga_v if par == 0 else gb_v
                        nxt = gb_v if par == 0 else ga_v
                        nch = jnp.where(ch + 1 < GS2, ch + 1, 0)
                        pltpu.async_copy(h_hbm.at[ge1.at[nch]], nxt, sg)
                        drain_gather(gbuf)

                        @pl.when(ch >= 1)
                        def _():
                            drain_scatter()
                        scale(ch, gbuf, sb_v)
                        pltpu.async_copy(sb_v, hp_sh.at[ge0.at[ch]], ss,
                                         add=True)
                    return 0
                lax.fori_loop(0, GS2 // 2, chunk_pair, 0)
                drain_gather(ga_v)
                drain_scatter()
                return 0
            lax.fori_loop(0, ngrp, bgrp, 0)
        plsc.subcore_barrier()
        pltpu.sync_copy(hp_sh.at[pl.ds(sid * SEGH, SEGH)],
                        hp_hbm.at[cid, pl.ds(sid * SEGH, SEGH)])

    return kern(h_pad, bk0, bk1, bka, counts)


def kernel(input, adj, edge, p_h, edge_col, row_i, row_resort, new_h, W, a1, a2):
    x_pad = jnp.pad(input, ((0, NP - N), (0, 0)))
    h_pad, u1, w2n = _node_pre(x_pad, W, a1, a2)
    v1, u2 = _edge_pre(p_h, new_h, W, a1, a2)

    ec0_2d = edge_col[0].reshape(NW, NCH, CH)
    row_2d = row_i.reshape(NW, NCH, CH)
    rr_2d = row_resort.reshape(NW, NCH, CH)
    e0_2d = edge[0].reshape(NW, NCH, CH)
    e1_2d = edge[1].reshape(NW, NCH, CH)

    ex1_2d, seg_part = _sc_attn_col(u1, v1.reshape(NW, NCH, CH), ec0_2d, row_2d)
    z_2d = _sc_softmax_div(seg_part, ex1_2d, row_2d)
    ee_2d, rs_part = _sc_edge_row(w2n, u2, z_2d.reshape(E), rr_2d, e1_2d, e0_2d)
    att_2d, bk0, bk1, bka, counts = _sc_partition(ee_2d, e0_2d, e1_2d,
                                                  rs_part)
    hp_halves = _sc_spmm(h_pad, bk0, bk1, bka, counts)

    h_prime = _combine(hp_halves.reshape(NP, D)[:N])
    return h_prime, edge, att_2d.reshape(E, 1)
